# baseline (speedup 1.0000x reference)
import functools

import jax
import jax.numpy as jnp
from jax import lax
from jax.experimental import pallas as pl
from jax.experimental.pallas import tpu as pltpu
from jax.experimental.pallas import tpu_sc as plsc

K = 16
DILATION = 2
KSEL = K * DILATION

NC = 2
NS = 16
NW = NC * NS
INT_MAX = 2147483647


def _dist_body(xl_ref, xr_ref, d_ref, g_ref, g16_ref):
    a = xl_ref[0]
    b = xr_ref[0]
    mm = jax.lax.dot_general(
        a, b, (((1,), (1,)), ((), ())), preferred_element_type=jnp.float32
    )
    asq = jnp.sum(a * a, axis=1, keepdims=True)
    bsq = jnp.sum(b * b, axis=1, keepdims=True)
    d = (asq + (-2.0 * mm)) + bsq.T
    d_ref[0] = d
    bi, n = d.shape
    g16 = jnp.min(d.reshape(bi, n // 16, 16), axis=2)
    g16_ref[0] = g16
    g_ref[0] = jnp.min(g16.reshape(bi, 32, n // 512), axis=2)


@functools.partial(jax.jit, static_argnames=("bi",))
def _pairwise_dist(xt, bi=256):
    B, N, C = xt.shape
    grid = (B, N // bi)
    return pl.pallas_call(
        _dist_body,
        grid=grid,
        in_specs=[
            pl.BlockSpec((1, bi, C), lambda b, i: (b, i, 0)),
            pl.BlockSpec((1, N, C), lambda b, i: (b, 0, 0)),
        ],
        out_specs=[
            pl.BlockSpec((1, bi, N), lambda b, i: (b, i, 0)),
            pl.BlockSpec((1, bi, 32), lambda b, i: (b, i, 0)),
            pl.BlockSpec((1, bi, N // 16), lambda b, i: (b, i, 0)),
        ],
        out_shape=[
            jax.ShapeDtypeStruct((B, N, N), jnp.float32),
            jax.ShapeDtypeStruct((B, N, 32), jnp.float32),
            jax.ShapeDtypeStruct((B, N, N // 16), jnp.float32),
        ],
        compiler_params=pltpu.CompilerParams(
            dimension_semantics=("parallel", "arbitrary"),
        ),
    )(xt, xt)


def _make_select(num_rows, n):
    rpw = num_rows // NW
    nvreg = n // 16
    cap = 1024 + 32
    mesh = plsc.VectorSubcoreMesh(
        core_axis_name="c", subcore_axis_name="s", num_cores=NC, num_subcores=NS
    )

    @functools.partial(
        pl.kernel,
        out_type=jax.ShapeDtypeStruct((num_rows * K,), jnp.int32),
        mesh=mesh,
        compiler_params=pltpu.CompilerParams(needs_layout_passes=False),
        scratch_types=[
            pltpu.VMEM((n,), jnp.float32),
            pltpu.VMEM((n,), jnp.float32),
            pltpu.VMEM((n // 16,), jnp.float32),
            pltpu.VMEM((n // 16,), jnp.float32),
            pltpu.VMEM((rpw * 32,), jnp.float32),
            pltpu.VMEM((n // 16 + 16,), jnp.int32),
            pltpu.VMEM((cap,), jnp.int32),
            pltpu.VMEM((32,), jnp.int32),
            pltpu.VMEM((rpw * K,), jnp.int32),
            pltpu.SemaphoreType.DMA,
            pltpu.SemaphoreType.DMA,
        ],
    )
    def select(dist_hbm, gmins_hbm, g16_hbm, out_hbm, buf_a, buf_b, g16a,
               g16b, gall, glist, cand, obuf, oall, sem_a, sem_b):
        wid = lax.axis_index("s") * NC + lax.axis_index("c")
        base = wid * rpw
        iota16 = lax.broadcasted_iota(jnp.int32, (16,), 0)
        inf_v = jnp.full((16,), jnp.inf, jnp.float32)
        imax_v = jnp.full((16,), INT_MAX, jnp.int32)
        ngrp = n // 16

        pltpu.sync_copy(gmins_hbm.at[pl.ds(base * 32, rpw * 32)], gall)
        pltpu.async_copy(dist_hbm.at[base], buf_a, sem_a)
        pltpu.async_copy(g16_hbm.at[base], g16a, sem_a)

        def process(buf, g16, r):
            ga = gall[pl.ds(r * 32, 16)]
            gb = gall[pl.ds(r * 32 + 16, 16)]
            sa = jnp.sort(ga)
            sb = jnp.sort(gb)
            hi = jnp.sort(jnp.maximum(sa, lax.rev(sb, (0,))))
            theta_t = hi[11]
            theta_l = hi[15]
            theta_tv = jnp.full((16,), theta_t, jnp.float32)
            theta_lv = jnp.full((16,), theta_l, jnp.float32)

            def grp_body(g, gcnt):
                gm = g16[pl.ds(g * 16, 16)]
                m = gm <= theta_tv
                plsc.store_compressed(
                    glist.at[pl.ds(gcnt, 16)], iota16 + g * 16, mask=m
                )
                return gcnt + plsc.all_reduce_population_count(m)[0]

            gcnt = lax.fori_loop(0, ngrp // 16, grp_body, jnp.int32(0),
                                 unroll=8)

            nbat = (gcnt + 15) >> 4
            gcnt_v = jnp.full((16,), gcnt, jnp.int32)

            def bat_body(t, cnt):
                gv = glist[pl.ds(t * 16, 16)]
                gvalid = (iota16 + t * 16) < gcnt_v
                gbase = jnp.minimum(jnp.maximum(gv, 0), ngrp - 1) * 16
                for j in range(16):
                    idxj = gbase + j
                    xj = plsc.load_gather(buf, [idxj])
                    mj = (xj <= theta_tv) & gvalid
                    plsc.store_compressed(
                        cand.at[pl.ds(cnt, 16)], idxj, mask=mj
                    )
                    cnt = cnt + plsc.all_reduce_population_count(mj)[0]
                return cnt

            cnt_t = lax.fori_loop(0, nbat, bat_body, jnp.int32(0))

            def full_scan(_):
                def scan_body(g, cnt):
                    x = buf[pl.ds(g * 16, 16)]
                    m = x <= theta_lv
                    plsc.store_compressed(
                        cand.at[pl.ds(cnt, 16)], iota16 + g * 16, mask=m
                    )
                    return cnt + plsc.all_reduce_population_count(m)[0]

                return lax.fori_loop(0, nvreg, scan_body, jnp.int32(0),
                                     unroll=8)

            cnt = lax.cond(cnt_t < 32, full_scan, lambda _: cnt_t, 0)

            nb = (cnt + 15) >> 4
            cnt_v = jnp.full((16,), cnt, jnp.int32)

            def sel_body(t, carry):
                c0v, c0i, c1v, c1i = carry
                off = t * 16
                idx = cand[pl.ds(off, 16)]
                valid = (iota16 + off) < cnt_v
                idx_safe = jnp.minimum(jnp.maximum(idx, 0), n - 1)
                vals = plsc.load_gather(buf, [idx_safe])
                vals = jnp.where(valid, vals, inf_v)
                idxm = jnp.where(valid, idx, imax_v)
                sv, si = plsc.sort_key_val(vals, idxm)
                rv, ri = lax.rev(sv, (0,)), lax.rev(si, (0,))
                lt = (rv < c1v) | ((rv == c1v) & (ri < c1i))
                lov = jnp.where(lt, rv, c1v)
                loi = jnp.where(lt, ri, c1i)
                lov, loi = plsc.sort_key_val(lov, loi)
                rv2, ri2 = lax.rev(lov, (0,)), lax.rev(loi, (0,))
                lt2 = (rv2 < c0v) | ((rv2 == c0v) & (ri2 < c0i))
                n0v = jnp.where(lt2, rv2, c0v)
                n0i = jnp.where(lt2, ri2, c0i)
                n1v = jnp.where(lt2, c0v, rv2)
                n1i = jnp.where(lt2, c0i, ri2)
                n0v, n0i = plsc.sort_key_val(n0v, n0i)
                n1v, n1i = plsc.sort_key_val(n1v, n1i)
                return n0v, n0i, n1v, n1i

            init = (inf_v, imax_v, inf_v, imax_v)
            _, c0i, _, c1i = lax.fori_loop(0, nb, sel_body, init)
            obuf[pl.ds(0, 16)] = c0i
            obuf[pl.ds(16, 16)] = c1i
            oall[pl.ds(r * K, 16)] = plsc.load_gather(obuf, [iota16 * 2])

        def outer(k2, carry):
            r0 = 2 * k2
            row = base + r0
            pltpu.async_copy(dist_hbm.at[row + 1], buf_b, sem_b)
            pltpu.async_copy(g16_hbm.at[row + 1], g16b, sem_b)
            pltpu.make_async_copy(dist_hbm.at[row], buf_a, sem_a).wait()
            pltpu.make_async_copy(g16_hbm.at[row], g16a, sem_a).wait()
            process(buf_a, g16a, r0)

            @pl.when(r0 + 2 < rpw)
            def _():
                pltpu.async_copy(dist_hbm.at[row + 2], buf_a, sem_a)
                pltpu.async_copy(g16_hbm.at[row + 2], g16a, sem_a)

            pltpu.make_async_copy(dist_hbm.at[row + 1], buf_b, sem_b).wait()
            pltpu.make_async_copy(g16_hbm.at[row + 1], g16b, sem_b).wait()
            process(buf_b, g16b, r0 + 1)
            return carry

        lax.fori_loop(0, rpw // 2, outer, jnp.int32(0))
        pltpu.sync_copy(oall, out_hbm.at[pl.ds(base * K, rpw * K)])

    return select


def kernel(x):
    B, C, N, _ = x.shape
    xt = jnp.squeeze(jnp.swapaxes(x, 2, 1), -1)
    dist, gmins, g16 = _pairwise_dist(xt)
    sel = _make_select(B * N, N)
    nn_idx = sel(
        dist.reshape(B * N, N),
        gmins.reshape(B * N * 32),
        g16.reshape(B * N, N // 16),
    )
    nn_idx = nn_idx.reshape(B, N, K)
    center = jnp.broadcast_to(
        jnp.arange(N, dtype=nn_idx.dtype)[None, :, None], (B, N, K)
    )
    return jnp.stack((nn_idx, center), axis=0)


# speedup vs baseline: 2.7790x; 2.7790x over previous
import functools

import jax
import jax.numpy as jnp
from jax import lax
from jax.experimental import pallas as pl
from jax.experimental.pallas import tpu as pltpu
from jax.experimental.pallas import tpu_sc as plsc

K = 16
DILATION = 2
KSEL = K * DILATION

NC = 2
NS = 16
NW = NC * NS
INT_MAX = 2147483647


def _dist_body(xl_ref, xr_ref, d_ref, g16_ref):
    a = xl_ref[0]
    b = xr_ref[0]
    mm = jax.lax.dot_general(
        a, b, (((1,), (1,)), ((), ())), preferred_element_type=jnp.float32
    )
    asq = jnp.sum(a * a, axis=1, keepdims=True)
    bsq = jnp.sum(b * b, axis=1, keepdims=True)
    d = (asq + (-2.0 * mm)) + bsq.T
    d_ref[0] = d
    bi, n = d.shape
    g16_ref[0] = jnp.min(d.reshape(bi, n // 256, 256), axis=1)


@functools.partial(jax.jit, static_argnames=("bi",))
def _pairwise_dist(xt, bi=256):
    B, N, C = xt.shape
    grid = (B, N // bi)
    return pl.pallas_call(
        _dist_body,
        grid=grid,
        in_specs=[
            pl.BlockSpec((1, bi, C), lambda b, i: (b, i, 0)),
            pl.BlockSpec((1, N, C), lambda b, i: (b, 0, 0)),
        ],
        out_specs=[
            pl.BlockSpec((1, bi, N), lambda b, i: (b, i, 0)),
            pl.BlockSpec((1, bi, 256), lambda b, i: (b, i, 0)),
        ],
        out_shape=[
            jax.ShapeDtypeStruct((B, N, N), jnp.float32),
            jax.ShapeDtypeStruct((B, N, 256), jnp.float32),
        ],
        compiler_params=pltpu.CompilerParams(
            dimension_semantics=("parallel", "arbitrary"),
        ),
    )(xt, xt)


def _make_select(num_rows, n):
    rpw = num_rows // NW
    nvreg = n // 16
    cap = 1024 + 32
    mesh = plsc.VectorSubcoreMesh(
        core_axis_name="c", subcore_axis_name="s", num_cores=NC, num_subcores=NS
    )

    @functools.partial(
        pl.kernel,
        out_type=jax.ShapeDtypeStruct((num_rows * K,), jnp.int32),
        mesh=mesh,
        compiler_params=pltpu.CompilerParams(needs_layout_passes=False),
        scratch_types=[
            pltpu.VMEM((n,), jnp.float32),
            pltpu.VMEM((n,), jnp.float32),
            pltpu.VMEM((256,), jnp.float32),
            pltpu.VMEM((256,), jnp.float32),
            pltpu.VMEM((256 + 16,), jnp.int32),
            pltpu.VMEM((cap,), jnp.int32),
            pltpu.VMEM((32,), jnp.int32),
            pltpu.VMEM((rpw * K,), jnp.int32),
            pltpu.SemaphoreType.DMA,
            pltpu.SemaphoreType.DMA,
        ],
    )
    def select(dist_hbm, g16_hbm, out_hbm, buf_a, buf_b, g16a,
               g16b, glist, cand, obuf, oall, sem_a, sem_b):
        wid = lax.axis_index("s") * NC + lax.axis_index("c")
        base = wid * rpw
        iota16 = lax.broadcasted_iota(jnp.int32, (16,), 0)
        inf_v = jnp.full((16,), jnp.inf, jnp.float32)
        imax_v = jnp.full((16,), INT_MAX, jnp.int32)
        ngrp = 256

        pltpu.async_copy(dist_hbm.at[base], buf_a, sem_a)
        pltpu.async_copy(g16_hbm.at[base], g16a, sem_a)

        def process(buf, g16, r):
            ga = g16[pl.ds(0, 16)]
            gb = g16[pl.ds(16, 16)]
            for v in range(2, 16, 2):
                ga = jnp.minimum(ga, g16[pl.ds(v * 16, 16)])
                gb = jnp.minimum(gb, g16[pl.ds(v * 16 + 16, 16)])
            sa = jnp.sort(ga)
            sb = jnp.sort(gb)
            hi = jnp.sort(jnp.maximum(sa, lax.rev(sb, (0,))))
            theta_t = hi[11]
            theta_l = hi[15]
            theta_tv = jnp.full((16,), theta_t, jnp.float32)
            theta_lv = jnp.full((16,), theta_l, jnp.float32)

            def grp_body(g, gcnt):
                gm = g16[pl.ds(g * 16, 16)]
                m = gm <= theta_tv
                plsc.store_compressed(
                    glist.at[pl.ds(gcnt, 16)], iota16 + g * 16, mask=m
                )
                return gcnt + plsc.all_reduce_population_count(m)[0]

            gcnt = lax.fori_loop(0, ngrp // 16, grp_body, jnp.int32(0),
                                 unroll=8)

            nbat = (gcnt + 15) >> 4
            gcnt_v = jnp.full((16,), gcnt, jnp.int32)

            def bat_body(t, cnt):
                gv = glist[pl.ds(t * 16, 16)]
                gvalid = (iota16 + t * 16) < gcnt_v
                gbase = jnp.minimum(jnp.maximum(gv, 0), ngrp - 1)
                for j in range(16):
                    idxj = gbase + j * 256
                    xj = plsc.load_gather(buf, [idxj])
                    mj = (xj <= theta_tv) & gvalid
                    plsc.store_compressed(
                        cand.at[pl.ds(cnt, 16)], idxj, mask=mj
                    )
                    cnt = cnt + plsc.all_reduce_population_count(mj)[0]
                return cnt

            cnt_t = lax.fori_loop(0, nbat, bat_body, jnp.int32(0))

            def full_scan(_):
                def scan_body(g, cnt):
                    x = buf[pl.ds(g * 16, 16)]
                    m = x <= theta_lv
                    plsc.store_compressed(
                        cand.at[pl.ds(cnt, 16)], iota16 + g * 16, mask=m
                    )
                    return cnt + plsc.all_reduce_population_count(m)[0]

                return lax.fori_loop(0, nvreg, scan_body, jnp.int32(0),
                                     unroll=8)

            cnt = lax.cond(cnt_t < 32, full_scan, lambda _: cnt_t, 0)

            nb = (cnt + 15) >> 4
            cnt_v = jnp.full((16,), cnt, jnp.int32)

            def sel_body(t, carry):
                c0v, c0i, c1v, c1i = carry
                off = t * 16
                idx = cand[pl.ds(off, 16)]
                valid = (iota16 + off) < cnt_v
                idx_safe = jnp.minimum(jnp.maximum(idx, 0), n - 1)
                vals = plsc.load_gather(buf, [idx_safe])
                vals = jnp.where(valid, vals, inf_v)
                idxm = jnp.where(valid, idx, imax_v)
                sv, si = plsc.sort_key_val(vals, idxm)
                rv, ri = lax.rev(sv, (0,)), lax.rev(si, (0,))
                lt = (rv < c1v) | ((rv == c1v) & (ri < c1i))
                lov = jnp.where(lt, rv, c1v)
                loi = jnp.where(lt, ri, c1i)
                lov, loi = plsc.sort_key_val(lov, loi)
                rv2, ri2 = lax.rev(lov, (0,)), lax.rev(loi, (0,))
                lt2 = (rv2 < c0v) | ((rv2 == c0v) & (ri2 < c0i))
                n0v = jnp.where(lt2, rv2, c0v)
                n0i = jnp.where(lt2, ri2, c0i)
                n1v = jnp.where(lt2, c0v, rv2)
                n1i = jnp.where(lt2, c0i, ri2)
                n0v, n0i = plsc.sort_key_val(n0v, n0i)
                n1v, n1i = plsc.sort_key_val(n1v, n1i)
                return n0v, n0i, n1v, n1i

            init = (inf_v, imax_v, inf_v, imax_v)
            _, c0i, _, c1i = lax.fori_loop(0, nb, sel_body, init)
            obuf[pl.ds(0, 16)] = c0i
            obuf[pl.ds(16, 16)] = c1i
            oall[pl.ds(r * K, 16)] = plsc.load_gather(obuf, [iota16 * 2])

        def outer(k2, carry):
            r0 = 2 * k2
            row = base + r0
            pltpu.async_copy(dist_hbm.at[row + 1], buf_b, sem_b)
            pltpu.async_copy(g16_hbm.at[row + 1], g16b, sem_b)
            pltpu.make_async_copy(dist_hbm.at[row], buf_a, sem_a).wait()
            pltpu.make_async_copy(g16_hbm.at[row], g16a, sem_a).wait()
            process(buf_a, g16a, r0)

            @pl.when(r0 + 2 < rpw)
            def _():
                pltpu.async_copy(dist_hbm.at[row + 2], buf_a, sem_a)
                pltpu.async_copy(g16_hbm.at[row + 2], g16a, sem_a)

            pltpu.make_async_copy(dist_hbm.at[row + 1], buf_b, sem_b).wait()
            pltpu.make_async_copy(g16_hbm.at[row + 1], g16b, sem_b).wait()
            process(buf_b, g16b, r0 + 1)
            return carry

        lax.fori_loop(0, rpw // 2, outer, jnp.int32(0))
        pltpu.sync_copy(oall, out_hbm.at[pl.ds(base * K, rpw * K)])

    return select


def kernel(x):
    B, C, N, _ = x.shape
    xt = jnp.squeeze(jnp.swapaxes(x, 2, 1), -1)
    dist, g16 = _pairwise_dist(xt)
    sel = _make_select(B * N, N)
    nn_idx = sel(dist.reshape(B * N, N), g16.reshape(B * N, 256))
    nn_idx = nn_idx.reshape(B, N, K)
    center = jnp.broadcast_to(
        jnp.arange(N, dtype=nn_idx.dtype)[None, :, None], (B, N, K)
    )
    return jnp.stack((nn_idx, center), axis=0)


# speedup vs baseline: 3.1821x; 1.1450x over previous
import functools

import jax
import jax.numpy as jnp
from jax import lax
from jax.experimental import pallas as pl
from jax.experimental.pallas import tpu as pltpu
from jax.experimental.pallas import tpu_sc as plsc

K = 16
DILATION = 2
KSEL = K * DILATION

NC = 2
NS = 16
NW = NC * NS
INT_MAX = 2147483647


def _dist_body(xl_ref, xr_ref, d_ref, g16_ref):
    a = xl_ref[0]
    b = xr_ref[0]
    mm = jax.lax.dot_general(
        a, b, (((1,), (1,)), ((), ())), preferred_element_type=jnp.float32
    )
    asq = jnp.sum(a * a, axis=1, keepdims=True)
    bsq = jnp.sum(b * b, axis=1, keepdims=True)
    d = (asq + (-2.0 * mm)) + bsq.T
    d_ref[0] = d
    bi, n = d.shape
    g16_ref[0] = jnp.min(d.reshape(bi, n // 256, 256), axis=1)


@functools.partial(jax.jit, static_argnames=("bi",))
def _pairwise_dist(xt, bi=256):
    B, N, C = xt.shape
    grid = (B, N // bi)
    return pl.pallas_call(
        _dist_body,
        grid=grid,
        in_specs=[
            pl.BlockSpec((1, bi, C), lambda b, i: (b, i, 0)),
            pl.BlockSpec((1, N, C), lambda b, i: (b, 0, 0)),
        ],
        out_specs=[
            pl.BlockSpec((1, bi, N), lambda b, i: (b, i, 0)),
            pl.BlockSpec((1, bi, 256), lambda b, i: (b, i, 0)),
        ],
        out_shape=[
            jax.ShapeDtypeStruct((B, N, N), jnp.float32),
            jax.ShapeDtypeStruct((B, N, 256), jnp.float32),
        ],
        compiler_params=pltpu.CompilerParams(
            dimension_semantics=("parallel", "arbitrary"),
        ),
    )(xt, xt)


def _make_select(num_rows, n):
    rpw = num_rows // NW
    nvreg = n // 16
    cap = 1024 + 32
    mesh = plsc.VectorSubcoreMesh(
        core_axis_name="c", subcore_axis_name="s", num_cores=NC, num_subcores=NS
    )

    @functools.partial(
        pl.kernel,
        out_type=jax.ShapeDtypeStruct((num_rows * K,), jnp.int32),
        mesh=mesh,
        compiler_params=pltpu.CompilerParams(needs_layout_passes=False),
        scratch_types=[
            pltpu.VMEM((n,), jnp.float32),
            pltpu.VMEM((n,), jnp.float32),
            pltpu.VMEM((256,), jnp.float32),
            pltpu.VMEM((256,), jnp.float32),
            pltpu.VMEM((256 + 16,), jnp.int32),
            pltpu.VMEM((cap,), jnp.int32),
            pltpu.VMEM((32,), jnp.int32),
            pltpu.VMEM((rpw * K,), jnp.int32),
            pltpu.SemaphoreType.DMA,
            pltpu.SemaphoreType.DMA,
        ],
    )
    def select(dist_hbm, g16_hbm, out_hbm, buf_a, buf_b, g16a,
               g16b, glist, cand, obuf, oall, sem_a, sem_b):
        wid = lax.axis_index("s") * NC + lax.axis_index("c")
        base = wid * rpw
        iota16 = lax.broadcasted_iota(jnp.int32, (16,), 0)
        inf_v = jnp.full((16,), jnp.inf, jnp.float32)
        imax_v = jnp.full((16,), INT_MAX, jnp.int32)
        ngrp = 256

        pltpu.async_copy(dist_hbm.at[base], buf_a, sem_a)
        pltpu.async_copy(g16_hbm.at[base], g16a, sem_a)

        def process(buf, g16, r):
            ga = g16[pl.ds(0, 16)]
            gb = g16[pl.ds(16, 16)]
            for v in range(2, 16, 2):
                ga = jnp.minimum(ga, g16[pl.ds(v * 16, 16)])
                gb = jnp.minimum(gb, g16[pl.ds(v * 16 + 16, 16)])
            sa = jnp.sort(ga)
            sb = jnp.sort(gb)
            hi = jnp.sort(jnp.maximum(sa, lax.rev(sb, (0,))))
            theta_t = hi[11]
            theta_l = hi[15]
            theta_tv = jnp.full((16,), theta_t, jnp.float32)
            theta_lv = jnp.full((16,), theta_l, jnp.float32)

            def grp_body(g, gcnt):
                gm = g16[pl.ds(g * 16, 16)]
                m = gm <= theta_tv
                plsc.store_compressed(
                    glist.at[pl.ds(gcnt, 16)], iota16 + g * 16, mask=m
                )
                return gcnt + plsc.all_reduce_population_count(m)[0]

            gcnt = lax.fori_loop(0, ngrp // 16, grp_body, jnp.int32(0),
                                 unroll=8)

            nbat = (gcnt + 15) >> 4
            gcnt_v = jnp.full((16,), gcnt, jnp.int32)

            def bat_body(t, cnt):
                gv = glist[pl.ds(t * 16, 16)]
                gvalid = (iota16 + t * 16) < gcnt_v
                gbase = jnp.minimum(jnp.maximum(gv, 0), ngrp - 1)
                for j in range(16):
                    idxj = gbase + j * 256
                    xj = plsc.load_gather(buf, [idxj])
                    mj = (xj <= theta_tv) & gvalid
                    plsc.store_compressed(
                        cand.at[pl.ds(cnt, 16)], idxj, mask=mj
                    )
                    cnt = cnt + plsc.all_reduce_population_count(mj)[0]
                return cnt

            cnt_t = lax.fori_loop(0, nbat, bat_body, jnp.int32(0))

            def full_scan(_):
                def scan_body(g, cnt):
                    x = buf[pl.ds(g * 16, 16)]
                    m = x <= theta_lv
                    plsc.store_compressed(
                        cand.at[pl.ds(cnt, 16)], iota16 + g * 16, mask=m
                    )
                    return cnt + plsc.all_reduce_population_count(m)[0]

                return lax.fori_loop(0, nvreg, scan_body, jnp.int32(0),
                                     unroll=8)

            cnt = lax.cond(cnt_t < 32, full_scan, lambda _: cnt_t, 0)

            nb = (cnt + 15) >> 4
            cnt_v = jnp.full((16,), cnt, jnp.int32)

            def sel_body(t, carry):
                c0v, c0i, c1v, c1i = carry
                off = t * 16
                idx = cand[pl.ds(off, 16)]
                valid = (iota16 + off) < cnt_v
                idx_safe = jnp.minimum(jnp.maximum(idx, 0), n - 1)
                vals = plsc.load_gather(buf, [idx_safe])
                vals = jnp.where(valid, vals, inf_v)
                idxm = jnp.where(valid, idx, imax_v)
                sv, si = plsc.sort_key_val(vals, idxm)
                rv, ri = lax.rev(sv, (0,)), lax.rev(si, (0,))
                lt = (rv < c1v) | ((rv == c1v) & (ri < c1i))
                lov = jnp.where(lt, rv, c1v)
                loi = jnp.where(lt, ri, c1i)
                lov, loi = plsc.sort_key_val(lov, loi)
                rv2, ri2 = lax.rev(lov, (0,)), lax.rev(loi, (0,))
                lt2 = (rv2 < c0v) | ((rv2 == c0v) & (ri2 < c0i))
                n0v = jnp.where(lt2, rv2, c0v)
                n0i = jnp.where(lt2, ri2, c0i)
                n1v = jnp.where(lt2, c0v, rv2)
                n1i = jnp.where(lt2, c0i, ri2)
                n0v, n0i = plsc.sort_key_val(n0v, n0i)
                n1v, n1i = plsc.sort_key_val(n1v, n1i)
                return n0v, n0i, n1v, n1i

            init = (inf_v, imax_v, inf_v, imax_v)
            _, c0i, _, c1i = lax.fori_loop(0, nb, sel_body, init)
            obuf[pl.ds(0, 16)] = c0i
            obuf[pl.ds(16, 16)] = c1i
            oall[pl.ds(r * K, 16)] = plsc.load_gather(obuf, [iota16 * 2])

        def outer(k2, carry):
            r0 = 2 * k2
            row = base + r0
            pltpu.async_copy(dist_hbm.at[row + 1], buf_b, sem_b)
            pltpu.async_copy(g16_hbm.at[row + 1], g16b, sem_b)
            pltpu.make_async_copy(dist_hbm.at[row], buf_a, sem_a).wait()
            pltpu.make_async_copy(g16_hbm.at[row], g16a, sem_a).wait()
            process(buf_a, g16a, r0)

            @pl.when(r0 + 2 < rpw)
            def _():
                pltpu.async_copy(dist_hbm.at[row + 2], buf_a, sem_a)
                pltpu.async_copy(g16_hbm.at[row + 2], g16a, sem_a)

            pltpu.make_async_copy(dist_hbm.at[row + 1], buf_b, sem_b).wait()
            pltpu.make_async_copy(g16_hbm.at[row + 1], g16b, sem_b).wait()
            process(buf_b, g16b, r0 + 1)
            return carry

        lax.fori_loop(0, rpw // 2, outer, jnp.int32(0))
        pltpu.sync_copy(oall, out_hbm.at[pl.ds(base * K, rpw * K)])

    return select


def kernel(x):
    B, C, N, _ = x.shape
    xt = jnp.squeeze(jnp.swapaxes(x, 2, 1), -1)
    sel = _make_select(N, N)
    parts = []
    for b in range(B):
        dist, g16 = _pairwise_dist(xt[b : b + 1])
        parts.append(sel(dist.reshape(N, N), g16.reshape(N, 256)))
    nn_idx = jnp.concatenate(parts)
    nn_idx = nn_idx.reshape(B, N, K)
    center = jnp.broadcast_to(
        jnp.arange(N, dtype=nn_idx.dtype)[None, :, None], (B, N, K)
    )
    return jnp.stack((nn_idx, center), axis=0)


# speedup vs baseline: 3.5348x; 1.1109x over previous
import functools

import jax
import jax.numpy as jnp
from jax import lax
from jax.experimental import pallas as pl
from jax.experimental.pallas import tpu as pltpu
from jax.experimental.pallas import tpu_sc as plsc

K = 16
DILATION = 2
KSEL = K * DILATION

NC = 2
NS = 16
NW = NC * NS
INT_MAX = 2147483647


def _dist_body(xl_ref, xr_ref, d_ref, g16_ref):
    a = xl_ref[0]
    b = xr_ref[0]
    mm = jax.lax.dot_general(
        a, b, (((1,), (1,)), ((), ())), preferred_element_type=jnp.float32
    )
    asq = jnp.sum(a * a, axis=1, keepdims=True)
    bsq = jnp.sum(b * b, axis=1, keepdims=True)
    d = (asq + (-2.0 * mm)) + bsq.T
    d_ref[0] = d
    bi, n = d.shape
    g16_ref[0] = jnp.min(d.reshape(bi, n // 256, 256), axis=1)


@functools.partial(jax.jit, static_argnames=("bi",))
def _pairwise_dist(xt, bi=256):
    B, N, C = xt.shape
    grid = (B, N // bi)
    return pl.pallas_call(
        _dist_body,
        grid=grid,
        in_specs=[
            pl.BlockSpec((1, bi, C), lambda b, i: (b, i, 0)),
            pl.BlockSpec((1, N, C), lambda b, i: (b, 0, 0)),
        ],
        out_specs=[
            pl.BlockSpec((1, bi, N), lambda b, i: (b, i, 0)),
            pl.BlockSpec((1, bi, 256), lambda b, i: (b, i, 0)),
        ],
        out_shape=[
            jax.ShapeDtypeStruct((B, N, N), jnp.float32),
            jax.ShapeDtypeStruct((B, N, 256), jnp.float32),
        ],
        compiler_params=pltpu.CompilerParams(
            dimension_semantics=("parallel", "arbitrary"),
        ),
    )(xt, xt)


def _make_select(num_rows, n):
    rpw = num_rows // NW
    nvreg = n // 16
    cap = 1024 + 32
    mesh = plsc.VectorSubcoreMesh(
        core_axis_name="c", subcore_axis_name="s", num_cores=NC, num_subcores=NS
    )

    @functools.partial(
        pl.kernel,
        out_type=jax.ShapeDtypeStruct((num_rows * K,), jnp.int32),
        mesh=mesh,
        compiler_params=pltpu.CompilerParams(needs_layout_passes=False),
        scratch_types=[
            pltpu.VMEM((n,), jnp.float32),
            pltpu.VMEM((n,), jnp.float32),
            pltpu.VMEM((256,), jnp.float32),
            pltpu.VMEM((256,), jnp.float32),
            pltpu.VMEM((256 + 16,), jnp.int32),
            pltpu.VMEM((cap,), jnp.int32),
            pltpu.VMEM((32,), jnp.int32),
            pltpu.VMEM((rpw * K,), jnp.int32),
            pltpu.SemaphoreType.DMA,
            pltpu.SemaphoreType.DMA,
        ],
    )
    def select(dist_hbm, g16_hbm, out_hbm, buf_a, buf_b, g16a,
               g16b, glist, cand, obuf, oall, sem_a, sem_b):
        wid = lax.axis_index("s") * NC + lax.axis_index("c")
        base = wid * rpw
        iota16 = lax.broadcasted_iota(jnp.int32, (16,), 0)
        inf_v = jnp.full((16,), jnp.inf, jnp.float32)
        imax_v = jnp.full((16,), INT_MAX, jnp.int32)
        ngrp = 256

        pltpu.async_copy(dist_hbm.at[base], buf_a, sem_a)
        pltpu.async_copy(g16_hbm.at[base], g16a, sem_a)

        def process(buf, g16, r):
            ga = g16[pl.ds(0, 16)]
            gb = g16[pl.ds(16, 16)]
            for v in range(2, 16, 2):
                ga = jnp.minimum(ga, g16[pl.ds(v * 16, 16)])
                gb = jnp.minimum(gb, g16[pl.ds(v * 16 + 16, 16)])
            sa = jnp.sort(ga)
            sb = jnp.sort(gb)
            hi = jnp.sort(jnp.maximum(sa, lax.rev(sb, (0,))))
            theta_t = hi[11]
            theta_l = hi[15]
            theta_tv = jnp.full((16,), theta_t, jnp.float32)
            theta_lv = jnp.full((16,), theta_l, jnp.float32)

            def grp_body(g, gcnt_v):
                gm = g16[pl.ds(g * 16, 16)]
                m = gm <= theta_tv
                pos = gcnt_v + plsc.cumsum(m.astype(jnp.int32)) - 1
                plsc.store_scatter(glist, [pos], iota16 + g * 16, mask=m)
                return gcnt_v + plsc.all_reduce_population_count(m)

            zero_v = jnp.zeros((16,), jnp.int32)
            gcnt_v = lax.fori_loop(0, ngrp // 16, grp_body, zero_v,
                                   unroll=8)
            gcnt = gcnt_v[0]

            nbat = (gcnt + 15) >> 4

            def bat_body(t, cnt_v):
                gv = glist[pl.ds(t * 16, 16)]
                gvalid = (iota16 + t * 16) < gcnt_v
                gbase = jnp.minimum(jnp.maximum(gv, 0), ngrp - 1)
                for j in range(16):
                    idxj = gbase + j * 256
                    xj = plsc.load_gather(buf, [idxj])
                    mj = (xj <= theta_tv) & gvalid
                    pos = cnt_v + plsc.cumsum(mj.astype(jnp.int32)) - 1
                    plsc.store_scatter(cand, [pos], idxj, mask=mj)
                    cnt_v = cnt_v + plsc.all_reduce_population_count(mj)
                return cnt_v

            cnt_tv = lax.fori_loop(0, nbat, bat_body, zero_v)
            cnt_t = cnt_tv[0]

            def full_scan(_):
                def scan_body(g, cnt_v):
                    x = buf[pl.ds(g * 16, 16)]
                    m = x <= theta_lv
                    pos = cnt_v + plsc.cumsum(m.astype(jnp.int32)) - 1
                    plsc.store_scatter(cand, [pos], iota16 + g * 16, mask=m)
                    return cnt_v + plsc.all_reduce_population_count(m)

                return lax.fori_loop(0, nvreg, scan_body, zero_v,
                                     unroll=8)[0]

            cnt = lax.cond(cnt_t < 32, full_scan, lambda _: cnt_t, 0)

            nb = (cnt + 15) >> 4
            cnt_v = jnp.full((16,), cnt, jnp.int32)

            def sel_body(t, carry):
                c0v, c0i, c1v, c1i = carry
                off = t * 16
                idx = cand[pl.ds(off, 16)]
                valid = (iota16 + off) < cnt_v
                idx_safe = jnp.minimum(jnp.maximum(idx, 0), n - 1)
                vals = plsc.load_gather(buf, [idx_safe])
                vals = jnp.where(valid, vals, inf_v)
                idxm = jnp.where(valid, idx, imax_v)
                sv, si = plsc.sort_key_val(vals, idxm)
                rv, ri = lax.rev(sv, (0,)), lax.rev(si, (0,))
                lt = (rv < c1v) | ((rv == c1v) & (ri < c1i))
                lov = jnp.where(lt, rv, c1v)
                loi = jnp.where(lt, ri, c1i)
                lov, loi = plsc.sort_key_val(lov, loi)
                rv2, ri2 = lax.rev(lov, (0,)), lax.rev(loi, (0,))
                lt2 = (rv2 < c0v) | ((rv2 == c0v) & (ri2 < c0i))
                n0v = jnp.where(lt2, rv2, c0v)
                n0i = jnp.where(lt2, ri2, c0i)
                n1v = jnp.where(lt2, c0v, rv2)
                n1i = jnp.where(lt2, c0i, ri2)
                n0v, n0i = plsc.sort_key_val(n0v, n0i)
                n1v, n1i = plsc.sort_key_val(n1v, n1i)
                return n0v, n0i, n1v, n1i

            init = (inf_v, imax_v, inf_v, imax_v)
            _, c0i, _, c1i = lax.fori_loop(0, nb, sel_body, init)
            obuf[pl.ds(0, 16)] = c0i
            obuf[pl.ds(16, 16)] = c1i
            oall[pl.ds(r * K, 16)] = plsc.load_gather(obuf, [iota16 * 2])

        def outer(k2, carry):
            r0 = 2 * k2
            row = base + r0
            pltpu.async_copy(dist_hbm.at[row + 1], buf_b, sem_b)
            pltpu.async_copy(g16_hbm.at[row + 1], g16b, sem_b)
            pltpu.make_async_copy(dist_hbm.at[row], buf_a, sem_a).wait()
            pltpu.make_async_copy(g16_hbm.at[row], g16a, sem_a).wait()
            process(buf_a, g16a, r0)

            @pl.when(r0 + 2 < rpw)
            def _():
                pltpu.async_copy(dist_hbm.at[row + 2], buf_a, sem_a)
                pltpu.async_copy(g16_hbm.at[row + 2], g16a, sem_a)

            pltpu.make_async_copy(dist_hbm.at[row + 1], buf_b, sem_b).wait()
            pltpu.make_async_copy(g16_hbm.at[row + 1], g16b, sem_b).wait()
            process(buf_b, g16b, r0 + 1)
            return carry

        lax.fori_loop(0, rpw // 2, outer, jnp.int32(0))
        pltpu.sync_copy(oall, out_hbm.at[pl.ds(base * K, rpw * K)])

    return select


def kernel(x):
    B, C, N, _ = x.shape
    xt = jnp.squeeze(jnp.swapaxes(x, 2, 1), -1)
    sel = _make_select(N, N)
    parts = []
    for b in range(B):
        dist, g16 = _pairwise_dist(xt[b : b + 1])
        parts.append(sel(dist.reshape(N, N), g16.reshape(N, 256)))
    nn_idx = jnp.concatenate(parts)
    nn_idx = nn_idx.reshape(B, N, K)
    center = jnp.broadcast_to(
        jnp.arange(N, dtype=nn_idx.dtype)[None, :, None], (B, N, K)
    )
    return jnp.stack((nn_idx, center), axis=0)


# speedup vs baseline: 3.8023x; 1.0757x over previous
import functools

import jax
import jax.numpy as jnp
from jax import lax
from jax.experimental import pallas as pl
from jax.experimental.pallas import tpu as pltpu
from jax.experimental.pallas import tpu_sc as plsc

K = 16
DILATION = 2
KSEL = K * DILATION

NC = 2
NS = 16
NW = NC * NS
INT_MAX = 2147483647


def _dist_body(xl_ref, xr_ref, d_ref, g16_ref):
    a = xl_ref[0]
    b = xr_ref[0]
    mm = jax.lax.dot_general(
        a, b, (((1,), (1,)), ((), ())), preferred_element_type=jnp.float32
    )
    asq = jnp.sum(a * a, axis=1, keepdims=True)
    bsq = jnp.sum(b * b, axis=1, keepdims=True)
    d = (asq + (-2.0 * mm)) + bsq.T
    d_ref[0] = d
    bi, n = d.shape
    g16_ref[0] = jnp.min(d.reshape(bi, n // 256, 256), axis=1)


@functools.partial(jax.jit, static_argnames=("bi",))
def _pairwise_dist(lhs, rhs, bi=256):
    M, C = lhs.shape
    N, _ = rhs.shape
    grid = (M // bi,)
    return pl.pallas_call(
        _dist_body,
        grid=grid,
        in_specs=[
            pl.BlockSpec((1, bi, C), lambda i: (0, i, 0)),
            pl.BlockSpec((1, N, C), lambda i: (0, 0, 0)),
        ],
        out_specs=[
            pl.BlockSpec((1, bi, N), lambda i: (0, i, 0)),
            pl.BlockSpec((1, bi, 256), lambda i: (0, i, 0)),
        ],
        out_shape=[
            jax.ShapeDtypeStruct((1, M, N), jnp.float32),
            jax.ShapeDtypeStruct((1, M, 256), jnp.float32),
        ],
        compiler_params=pltpu.CompilerParams(
            dimension_semantics=("arbitrary",),
        ),
    )(lhs[None], rhs[None])


def _make_select(num_rows, n):
    rpw = num_rows // NW
    nvreg = n // 16
    cap = 1024 + 32
    mesh = plsc.VectorSubcoreMesh(
        core_axis_name="c", subcore_axis_name="s", num_cores=NC, num_subcores=NS
    )

    @functools.partial(
        pl.kernel,
        out_type=jax.ShapeDtypeStruct((num_rows * K,), jnp.int32),
        mesh=mesh,
        compiler_params=pltpu.CompilerParams(needs_layout_passes=False),
        scratch_types=[
            pltpu.VMEM((n,), jnp.float32),
            pltpu.VMEM((n,), jnp.float32),
            pltpu.VMEM((256,), jnp.float32),
            pltpu.VMEM((256,), jnp.float32),
            pltpu.VMEM((256 + 16,), jnp.int32),
            pltpu.VMEM((cap,), jnp.int32),
            pltpu.VMEM((32,), jnp.int32),
            pltpu.VMEM((rpw * K,), jnp.int32),
            pltpu.SemaphoreType.DMA,
            pltpu.SemaphoreType.DMA,
        ],
    )
    def select(dist_hbm, g16_hbm, out_hbm, buf_a, buf_b, g16a,
               g16b, glist, cand, obuf, oall, sem_a, sem_b):
        wid = lax.axis_index("s") * NC + lax.axis_index("c")
        base = wid * rpw
        iota16 = lax.broadcasted_iota(jnp.int32, (16,), 0)
        inf_v = jnp.full((16,), jnp.inf, jnp.float32)
        imax_v = jnp.full((16,), INT_MAX, jnp.int32)
        ngrp = 256

        pltpu.async_copy(dist_hbm.at[base], buf_a, sem_a)
        pltpu.async_copy(g16_hbm.at[base], g16a, sem_a)

        def process(buf, g16, r):
            ga = g16[pl.ds(0, 16)]
            gb = g16[pl.ds(16, 16)]
            for v in range(2, 16, 2):
                ga = jnp.minimum(ga, g16[pl.ds(v * 16, 16)])
                gb = jnp.minimum(gb, g16[pl.ds(v * 16 + 16, 16)])
            sa = jnp.sort(ga)
            sb = jnp.sort(gb)
            hi = jnp.sort(jnp.maximum(sa, lax.rev(sb, (0,))))
            theta_t = hi[9]
            theta_l = hi[15]
            theta_tv = jnp.full((16,), theta_t, jnp.float32)
            theta_lv = jnp.full((16,), theta_l, jnp.float32)

            def grp_body(g, gcnt_v):
                gm = g16[pl.ds(g * 16, 16)]
                m = gm <= theta_tv
                pos = gcnt_v + plsc.cumsum(m.astype(jnp.int32)) - 1
                plsc.store_scatter(glist, [pos], iota16 + g * 16, mask=m)
                return gcnt_v + plsc.all_reduce_population_count(m)

            zero_v = jnp.zeros((16,), jnp.int32)
            gcnt_v = lax.fori_loop(0, ngrp // 16, grp_body, zero_v,
                                   unroll=8)
            gcnt = gcnt_v[0]

            nbat = (gcnt + 15) >> 4

            def bat_body(t, cnt_v):
                gv = glist[pl.ds(t * 16, 16)]
                gvalid = (iota16 + t * 16) < gcnt_v
                gbase = jnp.minimum(jnp.maximum(gv, 0), ngrp - 1)
                for j in range(16):
                    idxj = gbase + j * 256
                    xj = plsc.load_gather(buf, [idxj])
                    mj = (xj <= theta_tv) & gvalid
                    pos = cnt_v + plsc.cumsum(mj.astype(jnp.int32)) - 1
                    plsc.store_scatter(cand, [pos], idxj, mask=mj)
                    cnt_v = cnt_v + plsc.all_reduce_population_count(mj)
                return cnt_v

            cnt_tv = lax.fori_loop(0, nbat, bat_body, zero_v)
            cnt_t = cnt_tv[0]

            def full_scan(_):
                def scan_body(g, cnt_v):
                    x = buf[pl.ds(g * 16, 16)]
                    m = x <= theta_lv
                    pos = cnt_v + plsc.cumsum(m.astype(jnp.int32)) - 1
                    plsc.store_scatter(cand, [pos], iota16 + g * 16, mask=m)
                    return cnt_v + plsc.all_reduce_population_count(m)

                return lax.fori_loop(0, nvreg, scan_body, zero_v,
                                     unroll=8)[0]

            cnt = lax.cond(cnt_t < 32, full_scan, lambda _: cnt_t, 0)

            nb = (cnt + 15) >> 4
            cnt_v = jnp.full((16,), cnt, jnp.int32)

            def sel_body(t, carry):
                c0v, c0i, c1v, c1i = carry
                off = t * 16
                idx = cand[pl.ds(off, 16)]
                valid = (iota16 + off) < cnt_v
                idx_safe = jnp.minimum(jnp.maximum(idx, 0), n - 1)
                vals = plsc.load_gather(buf, [idx_safe])
                vals = jnp.where(valid, vals, inf_v)
                idxm = jnp.where(valid, idx, imax_v)
                sv, si = plsc.sort_key_val(vals, idxm)
                rv, ri = lax.rev(sv, (0,)), lax.rev(si, (0,))
                lt = (rv < c1v) | ((rv == c1v) & (ri < c1i))
                lov = jnp.where(lt, rv, c1v)
                loi = jnp.where(lt, ri, c1i)
                lov, loi = plsc.sort_key_val(lov, loi)
                rv2, ri2 = lax.rev(lov, (0,)), lax.rev(loi, (0,))
                lt2 = (rv2 < c0v) | ((rv2 == c0v) & (ri2 < c0i))
                n0v = jnp.where(lt2, rv2, c0v)
                n0i = jnp.where(lt2, ri2, c0i)
                n1v = jnp.where(lt2, c0v, rv2)
                n1i = jnp.where(lt2, c0i, ri2)
                n0v, n0i = plsc.sort_key_val(n0v, n0i)
                n1v, n1i = plsc.sort_key_val(n1v, n1i)
                return n0v, n0i, n1v, n1i

            init = (inf_v, imax_v, inf_v, imax_v)
            _, c0i, _, c1i = lax.fori_loop(0, nb, sel_body, init)
            obuf[pl.ds(0, 16)] = c0i
            obuf[pl.ds(16, 16)] = c1i
            oall[pl.ds(r * K, 16)] = plsc.load_gather(obuf, [iota16 * 2])

        def outer(k2, carry):
            r0 = 2 * k2
            row = base + r0
            pltpu.async_copy(dist_hbm.at[row + 1], buf_b, sem_b)
            pltpu.async_copy(g16_hbm.at[row + 1], g16b, sem_b)
            pltpu.make_async_copy(dist_hbm.at[row], buf_a, sem_a).wait()
            pltpu.make_async_copy(g16_hbm.at[row], g16a, sem_a).wait()
            process(buf_a, g16a, r0)

            @pl.when(r0 + 2 < rpw)
            def _():
                pltpu.async_copy(dist_hbm.at[row + 2], buf_a, sem_a)
                pltpu.async_copy(g16_hbm.at[row + 2], g16a, sem_a)

            pltpu.make_async_copy(dist_hbm.at[row + 1], buf_b, sem_b).wait()
            pltpu.make_async_copy(g16_hbm.at[row + 1], g16b, sem_b).wait()
            process(buf_b, g16b, r0 + 1)
            return carry

        lax.fori_loop(0, rpw // 2, outer, jnp.int32(0))
        pltpu.sync_copy(oall, out_hbm.at[pl.ds(base * K, rpw * K)])

    return select


def kernel(x):
    B, C, N, _ = x.shape
    xt = jnp.squeeze(jnp.swapaxes(x, 2, 1), -1)
    chunk = N // 2
    sel = _make_select(chunk, N)
    parts = []
    for b in range(B):
        for c in range(N // chunk):
            lhs = jax.lax.slice_in_dim(xt[b], c * chunk, (c + 1) * chunk)
            dist, g16 = _pairwise_dist(lhs, xt[b])
            parts.append(sel(dist.reshape(chunk, N), g16.reshape(chunk, 256)))
    nn_idx = jnp.concatenate(parts)
    nn_idx = nn_idx.reshape(B, N, K)
    center = jnp.broadcast_to(
        jnp.arange(N, dtype=nn_idx.dtype)[None, :, None], (B, N, K)
    )
    return jnp.stack((nn_idx, center), axis=0)


# speedup vs baseline: 5.0133x; 1.3185x over previous
import functools

import jax
import jax.numpy as jnp
from jax import lax
from jax.experimental import pallas as pl
from jax.experimental.pallas import tpu as pltpu
from jax.experimental.pallas import tpu_sc as plsc

K = 16
DILATION = 2
KSEL = K * DILATION

NC = 2
NS = 16
NW = NC * NS
INT_MAX = 2147483647


def _dist_body(xl_ref, xr_ref, d_ref, g16_ref, g16i_ref, g16s_ref):
    a = xl_ref[0]
    b = xr_ref[0]
    mm = jax.lax.dot_general(
        a, b, (((1,), (1,)), ((), ())), preferred_element_type=jnp.float32
    )
    asq = jnp.sum(a * a, axis=1, keepdims=True)
    bsq = jnp.sum(b * b, axis=1, keepdims=True)
    d = (asq + (-2.0 * mm)) + bsq.T
    d_ref[0] = d
    bi, n = d.shape
    nt = n // 256
    best = d[:, 0:256]
    bidx = jnp.zeros((bi, 256), jnp.int32)
    for t in range(1, nt):
        v = d[:, t * 256 : (t + 1) * 256]
        m = v < best
        best = jnp.where(m, v, best)
        bidx = jnp.where(m, t, bidx)
    best2 = jnp.full((bi, 256), jnp.inf, jnp.float32)
    for t in range(nt):
        v = d[:, t * 256 : (t + 1) * 256]
        vm = jnp.where(bidx == t, jnp.inf, v)
        best2 = jnp.minimum(best2, vm)
    g16_ref[0] = best
    g16i_ref[0] = bidx * 256 + lax.broadcasted_iota(jnp.int32, (bi, 256), 1)
    g16s_ref[0] = best2


@functools.partial(jax.jit, static_argnames=("bi",))
def _pairwise_dist(lhs, rhs, bi=256):
    M, C = lhs.shape
    N, _ = rhs.shape
    grid = (M // bi,)
    return pl.pallas_call(
        _dist_body,
        grid=grid,
        in_specs=[
            pl.BlockSpec((1, bi, C), lambda i: (0, i, 0)),
            pl.BlockSpec((1, N, C), lambda i: (0, 0, 0)),
        ],
        out_specs=[
            pl.BlockSpec((1, bi, N), lambda i: (0, i, 0)),
            pl.BlockSpec((1, bi, 256), lambda i: (0, i, 0)),
            pl.BlockSpec((1, bi, 256), lambda i: (0, i, 0)),
            pl.BlockSpec((1, bi, 256), lambda i: (0, i, 0)),
        ],
        out_shape=[
            jax.ShapeDtypeStruct((1, M, N), jnp.float32),
            jax.ShapeDtypeStruct((1, M, 256), jnp.float32),
            jax.ShapeDtypeStruct((1, M, 256), jnp.int32),
            jax.ShapeDtypeStruct((1, M, 256), jnp.float32),
        ],
        compiler_params=pltpu.CompilerParams(
            dimension_semantics=("arbitrary",),
        ),
    )(lhs[None], rhs[None])


def _make_select(num_rows, n):
    rpw = num_rows // NW
    nvreg = n // 16
    cap = 1024 + 32
    mesh = plsc.VectorSubcoreMesh(
        core_axis_name="c", subcore_axis_name="s", num_cores=NC, num_subcores=NS
    )

    @functools.partial(
        pl.kernel,
        out_type=jax.ShapeDtypeStruct((num_rows * K,), jnp.int32),
        mesh=mesh,
        compiler_params=pltpu.CompilerParams(needs_layout_passes=False),
        scratch_types=[
            pltpu.VMEM((n,), jnp.float32),
            pltpu.VMEM((n,), jnp.float32),
            pltpu.VMEM((256,), jnp.float32),
            pltpu.VMEM((256,), jnp.float32),
            pltpu.VMEM((256,), jnp.int32),
            pltpu.VMEM((256,), jnp.int32),
            pltpu.VMEM((256,), jnp.float32),
            pltpu.VMEM((256,), jnp.float32),
            pltpu.VMEM((256 + 16,), jnp.int32),
            pltpu.VMEM((cap,), jnp.int32),
            pltpu.VMEM((32,), jnp.int32),
            pltpu.VMEM((rpw * K,), jnp.int32),
            pltpu.SemaphoreType.DMA,
            pltpu.SemaphoreType.DMA,
        ],
    )
    def select(dist_hbm, g16_hbm, g16i_hbm, g16s_hbm, out_hbm, buf_a, buf_b,
               g16a, g16b, g16ia, g16ib, g16sa, g16sb, glist, cand, obuf,
               oall, sem_a, sem_b):
        wid = lax.axis_index("s") * NC + lax.axis_index("c")
        base = wid * rpw
        iota16 = lax.broadcasted_iota(jnp.int32, (16,), 0)
        inf_v = jnp.full((16,), jnp.inf, jnp.float32)
        imax_v = jnp.full((16,), INT_MAX, jnp.int32)
        ngrp = 256

        pltpu.async_copy(dist_hbm.at[base], buf_a, sem_a)
        pltpu.async_copy(g16_hbm.at[base], g16a, sem_a)
        pltpu.async_copy(g16i_hbm.at[base], g16ia, sem_a)
        pltpu.async_copy(g16s_hbm.at[base], g16sa, sem_a)

        def process(buf, g16, g16i, g16s, r):
            ga = g16[pl.ds(0, 16)]
            gb = g16[pl.ds(16, 16)]
            for v in range(2, 16, 2):
                ga = jnp.minimum(ga, g16[pl.ds(v * 16, 16)])
                gb = jnp.minimum(gb, g16[pl.ds(v * 16 + 16, 16)])
            sa = jnp.sort(ga)
            sb = jnp.sort(gb)
            hi = jnp.sort(jnp.maximum(sa, lax.rev(sb, (0,))))
            theta_t = hi[9]
            theta_l = hi[15]
            theta_tv = jnp.full((16,), theta_t, jnp.float32)
            theta_lv = jnp.full((16,), theta_l, jnp.float32)

            def grp_body(g, carry):
                cnt_v, dcnt_v = carry
                gm = g16[pl.ds(g * 16, 16)]
                gi = g16i[pl.ds(g * 16, 16)]
                g2 = g16s[pl.ds(g * 16, 16)]
                m = gm <= theta_tv
                pos = cnt_v + plsc.cumsum(m.astype(jnp.int32)) - 1
                plsc.store_scatter(cand, [pos], gi, mask=m)
                cnt_v = cnt_v + plsc.all_reduce_population_count(m)
                m2 = g2 <= theta_tv
                pos2 = dcnt_v + plsc.cumsum(m2.astype(jnp.int32)) - 1
                plsc.store_scatter(glist, [pos2], iota16 + g * 16, mask=m2)
                dcnt_v = dcnt_v + plsc.all_reduce_population_count(m2)
                return cnt_v, dcnt_v

            zero_v = jnp.zeros((16,), jnp.int32)
            acnt_v, dcnt_v = lax.fori_loop(
                0, ngrp // 16, grp_body, (zero_v, zero_v), unroll=8
            )

            ndbat = (dcnt_v[0] + 15) >> 4

            def deep_body(t, cnt_v):
                gv = glist[pl.ds(t * 16, 16)]
                gvalid = (iota16 + t * 16) < dcnt_v
                gbase = jnp.minimum(jnp.maximum(gv, 0), ngrp - 1)
                amv = plsc.load_gather(g16i, [gbase])
                for j in range(16):
                    idxj = gbase + j * 256
                    xj = plsc.load_gather(buf, [idxj])
                    mj = (xj <= theta_tv) & gvalid & (idxj != amv)
                    pos = cnt_v + plsc.cumsum(mj.astype(jnp.int32)) - 1
                    plsc.store_scatter(cand, [pos], idxj, mask=mj)
                    cnt_v = cnt_v + plsc.all_reduce_population_count(mj)
                return cnt_v

            cnt_tv = lax.fori_loop(0, ndbat, deep_body, acnt_v)
            cnt_t = cnt_tv[0]

            def full_scan(_):
                def scan_body(g, cnt_v):
                    x = buf[pl.ds(g * 16, 16)]
                    m = x <= theta_lv
                    pos = cnt_v + plsc.cumsum(m.astype(jnp.int32)) - 1
                    plsc.store_scatter(cand, [pos], iota16 + g * 16, mask=m)
                    return cnt_v + plsc.all_reduce_population_count(m)

                return lax.fori_loop(0, nvreg, scan_body, zero_v,
                                     unroll=8)[0]

            cnt = lax.cond(cnt_t < 32, full_scan, lambda _: cnt_t, 0)

            nb = (cnt + 15) >> 4
            cnt_v = jnp.full((16,), cnt, jnp.int32)

            def sel_body(t, carry):
                c0v, c0i, c1v, c1i = carry
                off = t * 16
                idx = cand[pl.ds(off, 16)]
                valid = (iota16 + off) < cnt_v
                idx_safe = jnp.minimum(jnp.maximum(idx, 0), n - 1)
                vals = plsc.load_gather(buf, [idx_safe])
                vals = jnp.where(valid, vals, inf_v)
                idxm = jnp.where(valid, idx, imax_v)
                sv, si = plsc.sort_key_val(vals, idxm)
                rv, ri = lax.rev(sv, (0,)), lax.rev(si, (0,))
                lt = (rv < c1v) | ((rv == c1v) & (ri < c1i))
                lov = jnp.where(lt, rv, c1v)
                loi = jnp.where(lt, ri, c1i)
                lov, loi = plsc.sort_key_val(lov, loi)
                rv2, ri2 = lax.rev(lov, (0,)), lax.rev(loi, (0,))
                lt2 = (rv2 < c0v) | ((rv2 == c0v) & (ri2 < c0i))
                n0v = jnp.where(lt2, rv2, c0v)
                n0i = jnp.where(lt2, ri2, c0i)
                n1v = jnp.where(lt2, c0v, rv2)
                n1i = jnp.where(lt2, c0i, ri2)
                n0v, n0i = plsc.sort_key_val(n0v, n0i)
                n1v, n1i = plsc.sort_key_val(n1v, n1i)
                return n0v, n0i, n1v, n1i

            init = (inf_v, imax_v, inf_v, imax_v)
            _, c0i, _, c1i = lax.fori_loop(0, nb, sel_body, init)
            obuf[pl.ds(0, 16)] = c0i
            obuf[pl.ds(16, 16)] = c1i
            oall[pl.ds(r * K, 16)] = plsc.load_gather(obuf, [iota16 * 2])

        def outer(k2, carry):
            r0 = 2 * k2
            row = base + r0
            pltpu.async_copy(dist_hbm.at[row + 1], buf_b, sem_b)
            pltpu.async_copy(g16_hbm.at[row + 1], g16b, sem_b)
            pltpu.async_copy(g16i_hbm.at[row + 1], g16ib, sem_b)
            pltpu.async_copy(g16s_hbm.at[row + 1], g16sb, sem_b)
            pltpu.make_async_copy(dist_hbm.at[row], buf_a, sem_a).wait()
            pltpu.make_async_copy(g16_hbm.at[row], g16a, sem_a).wait()
            pltpu.make_async_copy(g16i_hbm.at[row], g16ia, sem_a).wait()
            pltpu.make_async_copy(g16s_hbm.at[row], g16sa, sem_a).wait()
            process(buf_a, g16a, g16ia, g16sa, r0)

            @pl.when(r0 + 2 < rpw)
            def _():
                pltpu.async_copy(dist_hbm.at[row + 2], buf_a, sem_a)
                pltpu.async_copy(g16_hbm.at[row + 2], g16a, sem_a)
                pltpu.async_copy(g16i_hbm.at[row + 2], g16ia, sem_a)
                pltpu.async_copy(g16s_hbm.at[row + 2], g16sa, sem_a)

            pltpu.make_async_copy(dist_hbm.at[row + 1], buf_b, sem_b).wait()
            pltpu.make_async_copy(g16_hbm.at[row + 1], g16b, sem_b).wait()
            pltpu.make_async_copy(g16i_hbm.at[row + 1], g16ib, sem_b).wait()
            pltpu.make_async_copy(g16s_hbm.at[row + 1], g16sb, sem_b).wait()
            process(buf_b, g16b, g16ib, g16sb, r0 + 1)
            return carry

        lax.fori_loop(0, rpw // 2, outer, jnp.int32(0))
        pltpu.sync_copy(oall, out_hbm.at[pl.ds(base * K, rpw * K)])

    return select


def kernel(x):
    B, C, N, _ = x.shape
    xt = jnp.squeeze(jnp.swapaxes(x, 2, 1), -1)
    chunk = N // 2
    sel = _make_select(chunk, N)
    parts = []
    for b in range(B):
        for c in range(N // chunk):
            lhs = jax.lax.slice_in_dim(xt[b], c * chunk, (c + 1) * chunk)
            dist, g16, g16i, g16s = _pairwise_dist(lhs, xt[b])
            parts.append(
                sel(
                    dist.reshape(chunk, N),
                    g16.reshape(chunk, 256),
                    g16i.reshape(chunk, 256),
                    g16s.reshape(chunk, 256),
                )
            )
    nn_idx = jnp.concatenate(parts)
    nn_idx = nn_idx.reshape(B, N, K)
    center = jnp.broadcast_to(
        jnp.arange(N, dtype=nn_idx.dtype)[None, :, None], (B, N, K)
    )
    return jnp.stack((nn_idx, center), axis=0)
